# baseline (speedup 1.0000x reference)
import functools

import jax
import jax.numpy as jnp
from jax import lax
from jax.experimental import pallas as pl
from jax.experimental.pallas import tpu as pltpu
from jax.experimental.pallas import tpu_sc as plsc

_MAX_LEN = 200
_EMBED_DIM = 64
_FLAT = _MAX_LEN * _EMBED_DIM
_BATCH = 4096
_NC = 2
_NS = 16
_NW = _NC * _NS
_B_PER_W = _BATCH // _NW
_BLK = 32
_NCOPY = _B_PER_W // _BLK


def _sc_body(pe_hbm, out_hbm, shared, sem):
    sid = lax.axis_index("s")
    wid = sid * _NC + lax.axis_index("c")
    base = wid * _B_PER_W

    @pl.when(sid == 0)
    def _():
        for r in range(_BLK):
            pltpu.sync_copy(pe_hbm, shared.at[pl.ds(r, 1)])

    plsc.subcore_barrier()

    copies = [
        pltpu.make_async_copy(
            shared, out_hbm.at[pl.ds(base + j * _BLK, _BLK), :], sem
        )
        for j in range(_NCOPY)
    ]
    for c in copies:
        c.start()
    for c in copies:
        c.wait()


_sc_broadcast = functools.partial(
    pl.kernel,
    out_type=jax.ShapeDtypeStruct((_BATCH, _FLAT), jnp.float32),
    mesh=plsc.VectorSubcoreMesh(core_axis_name="c", subcore_axis_name="s"),
    scratch_types=[
        pltpu.MemorySpace.VMEM_SHARED((_BLK, _FLAT), jnp.float32),
        pltpu.SemaphoreType.DMA,
    ],
)(_sc_body)


def kernel(x, pe_weight):
    batch = x.shape[0]
    pe_flat = pe_weight.reshape(1, _FLAT)
    out = _sc_broadcast(pe_flat)
    return out.reshape(batch, _MAX_LEN, _EMBED_DIM)


# speedup vs baseline: 1.3408x; 1.3408x over previous
import jax
import jax.numpy as jnp
from jax.experimental import pallas as pl

_MAX_LEN = 200
_EMBED_DIM = 64
_FLAT = _MAX_LEN * _EMBED_DIM
_BB = 512


def _broadcast_body(pe_ref, o_ref):
    o_ref[...] = jnp.broadcast_to(pe_ref[...], o_ref.shape)


def kernel(x, pe_weight):
    batch = x.shape[0]
    pe_flat = pe_weight.reshape(1, _FLAT)
    out = pl.pallas_call(
        _broadcast_body,
        grid=(batch // _BB,),
        in_specs=[pl.BlockSpec((1, _FLAT), lambda i: (0, 0))],
        out_specs=pl.BlockSpec((_BB, _FLAT), lambda i: (i, 0)),
        out_shape=jax.ShapeDtypeStruct((batch, _FLAT), jnp.float32),
    )(pe_flat)
    return out.reshape(batch, _MAX_LEN, _EMBED_DIM)


# speedup vs baseline: 1.3411x; 1.0002x over previous
import jax
import jax.numpy as jnp
from jax.experimental import pallas as pl

_MAX_LEN = 200
_EMBED_DIM = 64
_FLAT = _MAX_LEN * _EMBED_DIM
_BB = 256


def _broadcast_body(pe_ref, o_ref):
    o_ref[...] = jnp.broadcast_to(pe_ref[...], o_ref.shape)


def kernel(x, pe_weight):
    batch = x.shape[0]
    pe_flat = pe_weight.reshape(1, _FLAT)
    out = pl.pallas_call(
        _broadcast_body,
        grid=(batch // _BB,),
        in_specs=[pl.BlockSpec((1, _FLAT), lambda i: (0, 0))],
        out_specs=pl.BlockSpec((_BB, _FLAT), lambda i: (i, 0)),
        out_shape=jax.ShapeDtypeStruct((batch, _FLAT), jnp.float32),
    )(pe_flat)
    return out.reshape(batch, _MAX_LEN, _EMBED_DIM)
